# baseline (speedup 1.0000x reference)
import jax
import jax.numpy as jnp
from jax.experimental import pallas as pl
from jax.experimental.pallas import tpu as pltpu

_P = 4


def _body(x_ref, w1_ref, b1_ref, w2_ref, b2d_ref, dec_ref, keep_ref):
    NB = x_ref.shape[0]
    G = 64 // _P
    h = jnp.maximum(
        jnp.dot(x_ref[...], w1_ref[...], preferred_element_type=jnp.float32)
        + b1_ref[...],
        0.0,
    )
    logits = jnp.dot(h, w2_ref[...], preferred_element_type=jnp.float32)
    diff = logits[:, _P:] - logits[:, :_P] + b2d_ref[0, 0]
    a3 = diff.reshape(NB // G, G, _P)
    m = jnp.max(jnp.max(a3, axis=2, keepdims=True), axis=1, keepdims=True)
    need = jnp.broadcast_to(m <= 0.0, a3.shape)
    first = (jax.lax.broadcasted_iota(jnp.int32, a3.shape, 1) == 0) & (
        jax.lax.broadcasted_iota(jnp.int32, a3.shape, 2) == 0
    )
    dec = jnp.where((a3 > 0.0) | (first & need), 1.0, 0.0)
    dec_ref[...] = dec.reshape(NB, _P)
    keep_ref[...] = jax.nn.sigmoid(diff)


def kernel(slots, W1, b1, W2, b2):
    B, K, D = slots.shape
    F = W1.shape[1]
    N = B * K // _P
    x4 = slots.reshape(N, _P * D)
    eye = jnp.eye(_P, dtype=slots.dtype)
    W1q = jnp.einsum("pq,df->pdqf", eye, W1).reshape(_P * D, _P * F)
    W2q = jnp.concatenate(
        [
            jnp.einsum("pq,f->pfq", eye, W2[:, 0]).reshape(_P * F, _P),
            jnp.einsum("pq,f->pfq", eye, W2[:, 1]).reshape(_P * F, _P),
        ],
        axis=1,
    )
    b1q = jnp.tile(b1, _P)
    b2d = (b2[1] - b2[0]).reshape(1, 1)

    NB = 2048
    NB = min(NB, N)
    grid = (N // NB,)
    dec4, keep4 = pl.pallas_call(
        _body,
        grid=grid,
        in_specs=[
            pl.BlockSpec((NB, _P * D), lambda i: (i, 0)),
            pl.BlockSpec((_P * D, _P * F), lambda i: (0, 0)),
            pl.BlockSpec((_P * F,), lambda i: (0,)),
            pl.BlockSpec((_P * F, 2 * _P), lambda i: (0, 0)),
            pl.BlockSpec(memory_space=pltpu.SMEM),
        ],
        out_specs=[
            pl.BlockSpec((NB, _P), lambda i: (i, 0)),
            pl.BlockSpec((NB, _P), lambda i: (i, 0)),
        ],
        out_shape=[
            jax.ShapeDtypeStruct((N, _P), jnp.float32),
            jax.ShapeDtypeStruct((N, _P), jnp.float32),
        ],
        compiler_params=pltpu.CompilerParams(
            dimension_semantics=("parallel",),
        ),
    )(x4, W1q, b1q, W2q, b2d)
    return (dec4.reshape(B, K), keep4.reshape(B, K))


# speedup vs baseline: 1.9132x; 1.9132x over previous
import jax
import jax.numpy as jnp
from jax.experimental import pallas as pl
from jax.experimental.pallas import tpu as pltpu

_C = 256


def _tn(lhs, rhs):
    return jax.lax.dot_general(
        lhs, rhs, (((0,), (0,)), ((), ())), preferred_element_type=jnp.float32
    )


def _body(x_ref, w1_ref, b1_ref, w2_ref, b2d_ref, i256_ref, dec_ref, keep_ref):
    TB, K, D = x_ref.shape
    N = TB * K
    x2 = x_ref[...].reshape(N, D)
    i256 = i256_ref[...]
    xT = jnp.concatenate(
        [_tn(x2[c * _C : (c + 1) * _C, :], i256) for c in range(N // _C)], axis=1
    )
    hT = jnp.maximum(_tn(w1_ref[...], xT) + b1_ref[...].reshape(D // 2, 1), 0.0)
    logitsT = _tn(w2_ref[...], hT)
    diffT = logitsT[1:2, :] - logitsT[0:1, :] + b2d_ref[0, 0]
    dm = diffT.reshape(N // 128, 128)
    lane = jax.lax.broadcasted_iota(jnp.int32, dm.shape, 1)
    left = lane < K
    neg = jnp.float32(-3.0e38)
    ml = jnp.max(jnp.where(left, dm, neg), axis=1, keepdims=True)
    mr = jnp.max(jnp.where(left, neg, dm), axis=1, keepdims=True)
    need = jnp.where(left, ml, mr) <= 0.0
    first = (lane == 0) | (lane == K)
    dec_ref[...] = jnp.where((dm > 0.0) | (first & need), 1.0, 0.0)
    keep_ref[...] = jax.nn.sigmoid(dm)


def kernel(slots, W1, b1, W2, b2):
    B, K, D = slots.shape
    F = W1.shape[1]
    TB = min(128, B)
    grid = (B // TB,)
    b2d = (b2[1] - b2[0]).reshape(1, 1)
    i256 = jnp.eye(_C, dtype=jnp.float32)
    dec, keep = pl.pallas_call(
        _body,
        grid=grid,
        in_specs=[
            pl.BlockSpec((TB, K, D), lambda i: (i, 0, 0)),
            pl.BlockSpec((D, F), lambda i: (0, 0)),
            pl.BlockSpec((F,), lambda i: (0,)),
            pl.BlockSpec((F, 2), lambda i: (0, 0)),
            pl.BlockSpec(memory_space=pltpu.SMEM),
            pl.BlockSpec((_C, _C), lambda i: (0, 0)),
        ],
        out_specs=[
            pl.BlockSpec((TB * K // 128, 128), lambda i: (i, 0)),
            pl.BlockSpec((TB * K // 128, 128), lambda i: (i, 0)),
        ],
        out_shape=[
            jax.ShapeDtypeStruct((B * K // 128, 128), jnp.float32),
            jax.ShapeDtypeStruct((B * K // 128, 128), jnp.float32),
        ],
        compiler_params=pltpu.CompilerParams(
            dimension_semantics=("parallel",),
        ),
    )(slots, W1, b1, W2, b2d, i256)
    return (dec.reshape(B, K), keep.reshape(B, K))
